# baseline (speedup 1.0000x reference)
import jax
import jax.numpy as jnp
from jax import lax
from jax.experimental import pallas as pl
from jax.experimental.pallas import tpu as pltpu
from jax.experimental.pallas import tpu_sc as plsc

VOCAB = 100002
DIM = 128
BATCH = 16384
C = 5
NC = 2
NS = 16
L = 16
NW = NC * NS
B_PER_W = BATCH // NW
CB = 32
N_CHUNKS = B_PER_W // CB
NG = CB // L


def _dots_body(tgt_hbm, ctx_hbm, ttab_hbm, ctab_hbm, out_hbm,
               idx_t, idx_c, rows_t2, rows_c2, out_v,
               sem_t0, sem_t1, sem_t2, sem_t3, sem_c0, sem_c1, sem_c2,
               sem_c3):
    wid = lax.axis_index("s") * NC + lax.axis_index("c")
    lane = lax.iota(jnp.int32, L)
    r5 = lane * C

    pltpu.sync_copy(tgt_hbm.at[pl.ds(wid * B_PER_W, B_PER_W)], idx_t)
    pltpu.sync_copy(ctx_hbm.at[pl.ds(wid * B_PER_W * C, B_PER_W * C)], idx_c)

    sem_t = (sem_t0, sem_t1, sem_t2, sem_t3)
    sem_c = (sem_c0, sem_c1, sem_c2, sem_c3)
    HALF = CB * C // 2

    def gather_descs(g, b):
        ot = pl.multiple_of(g * CB, 8)
        oc = pl.multiple_of(g * (CB * C), 8)
        dt = pltpu.make_async_copy(
            ttab_hbm.at[idx_t.at[pl.ds(ot, CB)]], rows_t2.at[b], sem_t[b])
        dc0 = pltpu.make_async_copy(
            ctab_hbm.at[idx_c.at[pl.ds(oc, HALF)]],
            rows_c2.at[b, pl.ds(0, HALF)], sem_c[b])
        dc1 = pltpu.make_async_copy(
            ctab_hbm.at[idx_c.at[pl.ds(oc + HALF, HALF)]],
            rows_c2.at[b, pl.ds(HALF, HALF)], sem_c[b])
        return dt, dc0, dc1

    def issue(g, b):
        for d in gather_descs(g, b):
            d.start()

    issue(0, 0)
    issue(1, 1)
    issue(2, 2)
    issue(3, 3)

    def do_chunk(g, b):
            dt, dc0, dc1 = gather_descs(g, b)
            dt.wait()
            dc0.wait()
            dc1.wait()

            @pl.loop(0, NG)
            def _groups(q):
                rows_t = rows_t2.at[b, pl.ds(q * L, L)]
                rows_c = rows_c2.at[b, pl.ds(q * (L * C), L * C)]

                @pl.loop(0, DIM,
                         init_carry=tuple(jnp.zeros((L,), jnp.float32)
                                          for _ in range(C)), unroll=2)
                def accs(e, carry):
                    a0, a1, a2, a3, a4 = carry
                    col = (lane + e) & (DIM - 1)
                    w = plsc.load_gather(rows_t, [lane, col])
                    a0 = a0 + w * plsc.load_gather(rows_c, [r5, col])
                    a1 = a1 + w * plsc.load_gather(rows_c, [r5 + 1, col])
                    a2 = a2 + w * plsc.load_gather(rows_c, [r5 + 2, col])
                    a3 = a3 + w * plsc.load_gather(rows_c, [r5 + 3, col])
                    a4 = a4 + w * plsc.load_gather(rows_c, [r5 + 4, col])
                    return a0, a1, a2, a3, a4

                ob = g * CB + q * L + lane
                for c in range(C):
                    plsc.store_scatter(
                        out_v, [jnp.full((L,), c, jnp.int32), ob], accs[c])

            @pl.when(g + 4 < N_CHUNKS)
            def _prefetch():
                issue(g + 4, b)

    @pl.loop(0, N_CHUNKS, step=4)
    def _chunks(ch):
        for b in range(4):
            do_chunk(ch + b, b)

    pltpu.sync_copy(out_v, out_hbm.at[:, pl.ds(wid * B_PER_W, B_PER_W)])


@jax.jit
def _dots(target_flat, context_flat, target_table, context_table):
    mesh = plsc.VectorSubcoreMesh(
        core_axis_name="c", subcore_axis_name="s",
        num_cores=NC, num_subcores=NS)
    return pl.kernel(
        _dots_body,
        out_type=jax.ShapeDtypeStruct((8, BATCH), jnp.float32),
        mesh=mesh,
        compiler_params=pltpu.CompilerParams(needs_layout_passes=False),
        scratch_types=[
            pltpu.VMEM((B_PER_W,), jnp.int32),
            pltpu.VMEM((B_PER_W * C,), jnp.int32),
            pltpu.VMEM((4, CB, DIM), jnp.float32),
            pltpu.VMEM((4, CB * C, DIM), jnp.float32),
            pltpu.VMEM((8, B_PER_W), jnp.float32),
            pltpu.SemaphoreType.DMA,
            pltpu.SemaphoreType.DMA,
            pltpu.SemaphoreType.DMA,
            pltpu.SemaphoreType.DMA,
            pltpu.SemaphoreType.DMA,
            pltpu.SemaphoreType.DMA,
            pltpu.SemaphoreType.DMA,
            pltpu.SemaphoreType.DMA,
        ],
    )(target_flat, context_flat, target_table, context_table)


def kernel(target, context, target_table, context_table):
    target_flat = target.reshape(-1).astype(jnp.int32)
    context_flat = context.reshape(-1).astype(jnp.int32)
    out = _dots(target_flat, context_flat, target_table, context_table)
    return out[:C].T


# speedup vs baseline: 1.0307x; 1.0307x over previous
import jax
import jax.numpy as jnp
from jax import lax
from jax.experimental import pallas as pl
from jax.experimental.pallas import tpu as pltpu
from jax.experimental.pallas import tpu_sc as plsc

VOCAB = 100002
DIM = 128
BATCH = 16384
C = 5
NC = 2
NS = 16
L = 16
NW = NC * NS
B_PER_W = BATCH // NW
CB = 32
N_CHUNKS = B_PER_W // CB
NG = CB // L


def _dots_body(tgt_hbm, ctx_hbm, ttab_hbm, ctab_hbm, out_hbm,
               idx_t, idx_c, rows_t2, rows_c2, out_v,
               sem_t0, sem_t1, sem_t2, sem_t3, sem_c0, sem_c1, sem_c2,
               sem_c3):
    wid = lax.axis_index("s") * NC + lax.axis_index("c")
    lane = lax.iota(jnp.int32, L)
    r5 = lane * C

    pltpu.sync_copy(tgt_hbm.at[pl.ds(wid * B_PER_W, B_PER_W)], idx_t)
    pltpu.sync_copy(ctx_hbm.at[pl.ds(wid * B_PER_W * C, B_PER_W * C)], idx_c)

    sem_t = (sem_t0, sem_t1, sem_t2, sem_t3)
    sem_c = (sem_c0, sem_c1, sem_c2, sem_c3)
    HALF = CB * C // 2

    def gather_descs(g, b):
        ot = pl.multiple_of(g * CB, 8)
        oc = pl.multiple_of(g * (CB * C), 8)
        dt = pltpu.make_async_copy(
            ttab_hbm.at[idx_t.at[pl.ds(ot, CB)]], rows_t2.at[b], sem_t[b])
        dc0 = pltpu.make_async_copy(
            ctab_hbm.at[idx_c.at[pl.ds(oc, HALF)]],
            rows_c2.at[b, pl.ds(0, HALF)], sem_c[b])
        dc1 = pltpu.make_async_copy(
            ctab_hbm.at[idx_c.at[pl.ds(oc + HALF, HALF)]],
            rows_c2.at[b, pl.ds(HALF, HALF)], sem_c[b])
        return dt, dc0, dc1

    def issue(g, b):
        for d in gather_descs(g, b):
            d.start()

    issue(0, 0)
    issue(1, 1)
    issue(2, 2)
    issue(3, 3)

    def do_chunk(g, b):
            dt, dc0, dc1 = gather_descs(g, b)
            dt.wait()
            dc0.wait()
            dc1.wait()

            @pl.loop(0, NG)
            def _groups(q):
                rows_t = rows_t2.at[b, pl.ds(q * L, L)]
                rows_c = rows_c2.at[b, pl.ds(q * (L * C), L * C)]

                @pl.loop(0, DIM,
                         init_carry=tuple(jnp.zeros((L,), jnp.float32)
                                          for _ in range(C)), unroll=4)
                def accs(e, carry):
                    a0, a1, a2, a3, a4 = carry
                    col = (lane + e) & (DIM - 1)
                    w = plsc.load_gather(rows_t, [lane, col])
                    a0 = a0 + w * plsc.load_gather(rows_c, [r5, col])
                    a1 = a1 + w * plsc.load_gather(rows_c, [r5 + 1, col])
                    a2 = a2 + w * plsc.load_gather(rows_c, [r5 + 2, col])
                    a3 = a3 + w * plsc.load_gather(rows_c, [r5 + 3, col])
                    a4 = a4 + w * plsc.load_gather(rows_c, [r5 + 4, col])
                    return a0, a1, a2, a3, a4

                ob = g * CB + q * L + lane
                for c in range(C):
                    plsc.store_scatter(
                        out_v, [jnp.full((L,), c, jnp.int32), ob], accs[c])

            @pl.when(g + 4 < N_CHUNKS)
            def _prefetch():
                issue(g + 4, b)

    @pl.loop(0, N_CHUNKS, step=4)
    def _chunks(ch):
        for b in range(4):
            do_chunk(ch + b, b)

    pltpu.sync_copy(out_v, out_hbm.at[:, pl.ds(wid * B_PER_W, B_PER_W)])


@jax.jit
def _dots(target_flat, context_flat, target_table, context_table):
    mesh = plsc.VectorSubcoreMesh(
        core_axis_name="c", subcore_axis_name="s",
        num_cores=NC, num_subcores=NS)
    return pl.kernel(
        _dots_body,
        out_type=jax.ShapeDtypeStruct((8, BATCH), jnp.float32),
        mesh=mesh,
        compiler_params=pltpu.CompilerParams(needs_layout_passes=False),
        scratch_types=[
            pltpu.VMEM((B_PER_W,), jnp.int32),
            pltpu.VMEM((B_PER_W * C,), jnp.int32),
            pltpu.VMEM((4, CB, DIM), jnp.float32),
            pltpu.VMEM((4, CB * C, DIM), jnp.float32),
            pltpu.VMEM((8, B_PER_W), jnp.float32),
            pltpu.SemaphoreType.DMA,
            pltpu.SemaphoreType.DMA,
            pltpu.SemaphoreType.DMA,
            pltpu.SemaphoreType.DMA,
            pltpu.SemaphoreType.DMA,
            pltpu.SemaphoreType.DMA,
            pltpu.SemaphoreType.DMA,
            pltpu.SemaphoreType.DMA,
        ],
    )(target_flat, context_flat, target_table, context_table)


def kernel(target, context, target_table, context_table):
    target_flat = target.reshape(-1).astype(jnp.int32)
    context_flat = context.reshape(-1).astype(jnp.int32)
    out = _dots(target_flat, context_flat, target_table, context_table)
    return out[:C].T
